# baseline (speedup 1.0000x reference)
import functools

import jax
import jax.numpy as jnp
from jax import lax
from jax.experimental import pallas as pl
from jax.experimental.pallas import tpu as pltpu
from jax.experimental.pallas import tpu_sc as plsc

EMB = 64
REPR = 32
NREL = 64


def _sc_gather2(hidx, tidx, node_emb, n_rows):
    nw, n_ch, ch = hidx.shape
    info = plsc.get_sparse_core_info()
    mesh = plsc.VectorSubcoreMesh(core_axis_name="c", subcore_axis_name="s")
    per_w = n_ch * ch
    out_t = jax.ShapeDtypeStruct((n_rows, EMB), jnp.float32)

    @functools.partial(
        pl.kernel,
        out_type=(out_t, out_t),
        mesh=mesh,
        scratch_types=[
            pltpu.VMEM((n_ch, ch), jnp.int32),
            pltpu.VMEM((n_ch, ch), jnp.int32),
            pltpu.VMEM((ch, EMB), jnp.float32),
            pltpu.VMEM((ch, EMB), jnp.float32),
            pltpu.SemaphoreType.DMA,
            pltpu.SemaphoreType.DMA,
        ],
        compiler_params=pltpu.CompilerParams(use_tc_tiling_on_sc=False),
    )
    def k(hidx_hbm, tidx_hbm, table_hbm, h_hbm, t_hbm,
          hidx_v, tidx_v, hrows, trows, sem_h, sem_t):
        wid = lax.axis_index("s") * info.num_cores + lax.axis_index("c")
        pltpu.sync_copy(hidx_hbm.at[wid], hidx_v)
        pltpu.sync_copy(tidx_hbm.at[wid], tidx_v)
        base = wid * per_w

        def body(c, carry):
            cph = pltpu.async_copy(table_hbm.at[hidx_v.at[c]], hrows, sem_h)
            cpt = pltpu.async_copy(table_hbm.at[tidx_v.at[c]], trows, sem_t)
            cph.wait()
            cpt.wait()
            pltpu.sync_copy(hrows, h_hbm.at[pl.ds(base + c * ch, ch)])
            pltpu.sync_copy(trows, t_hbm.at[pl.ds(base + c * ch, ch)])
            return carry

        lax.fori_loop(0, n_ch, body, 0)

    return k(hidx, tidx, node_emb)


def _tc_loss_sum(h, t, r_col, waug, block_e):
    e_total = h.shape[0]
    nblk = e_total // block_e
    ncol = NREL * REPR

    def body(h_ref, t_ref, r_ref, w_ref, out_ref):
        i = pl.program_id(0)
        d = h_ref[...] - t_ref[...]
        r = r_ref[...]
        oh = (lax.broadcasted_iota(jnp.int32, (block_e, NREL), 1) == r)
        dp = jnp.concatenate(
            [d.astype(jnp.bfloat16), oh.astype(jnp.bfloat16)], axis=1)
        t = jnp.dot(dp, w_ref[...], preferred_element_type=jnp.float32)
        colrel = lax.shift_right_logical(
            lax.broadcasted_iota(jnp.int32, (block_e, ncol), 1), 5)
        sel = jnp.where(colrel == r, t, 0.0)
        s = jnp.sum(sel * sel)

        @pl.when(i == 0)
        def _():
            out_ref[...] = jnp.zeros_like(out_ref)

        out_ref[...] += s

    out = pl.pallas_call(
        body,
        grid=(nblk,),
        in_specs=[
            pl.BlockSpec((block_e, EMB), lambda i: (i, 0)),
            pl.BlockSpec((block_e, EMB), lambda i: (i, 0)),
            pl.BlockSpec((block_e, 1), lambda i: (i, 0)),
            pl.BlockSpec((2 * EMB, ncol), lambda i: (0, 0)),
        ],
        out_specs=pl.BlockSpec((1, 1), lambda i: (0, 0)),
        out_shape=jax.ShapeDtypeStruct((1, 1), jnp.float32),
    )(h, t, r_col, waug)
    return out[0, 0]


def kernel(edge_index_t, edge_attr, node_emb, r_emb_w, r_proj_w):
    e_total = edge_index_t.shape[0]

    nw, ch = 32, 128
    n_ch = e_total // (nw * ch)
    hidx = edge_index_t[:, 0].reshape(nw, n_ch, ch)
    tidx = edge_index_t[:, 1].reshape(nw, n_ch, ch)

    h, t = _sc_gather2(hidx, tidx, node_emb, e_total)

    wt = r_proj_w.reshape(NREL, EMB, REPR).transpose(1, 0, 2).reshape(
        EMB, NREL * REPR)
    wtile = jnp.broadcast_to(r_emb_w[:, None, :], (NREL, NREL, REPR)).reshape(
        NREL, NREL * REPR)
    waug = jnp.concatenate([wt, wtile], axis=0).astype(jnp.bfloat16)

    r_col = edge_attr[:, 1:2]

    total = _tc_loss_sum(h, t, r_col, waug, block_e=512)
    return total / jnp.float32(e_total * REPR)
